# Initial kernel scaffold; baseline (speedup 1.0000x reference)
import jax
import jax.numpy as jnp
from jax.experimental import pallas as pl


def kernel(pred, labels):
    raise NotImplementedError("write your pallas kernel here")


# speedup vs baseline: 35.5464x; 35.5464x over previous
import functools

import jax
import jax.numpy as jnp
from jax import lax
from jax.experimental import pallas as pl
from jax.experimental.pallas import tpu as pltpu
from jax.experimental.pallas import tpu_sc as plsc

_B = 64
_C = 10000
_TOPN = 64
_NSUB = 16
_RPS = _B // _NSUB
_CHUNKS = _C // 16
_UNROLL = 25
_ROWW = 80


def _iota():
  return lax.iota(jnp.int32, 16)


def _splat_f(x):
  return jnp.zeros((16,), jnp.float32) + x


def _splat_i(x):
  return jnp.zeros((16,), jnp.int32) + x


def _vlog(v):
  ib = lax.bitcast_convert_type(v, jnp.int32)
  y = ib.astype(jnp.float32) * jnp.float32(8.262958405176314e-08) - jnp.float32(
      88.02969193111305)
  for _ in range(3):
    y = y - 1.0 + v * jnp.exp(-y)
  return y


def _sc_body(pred_hbm, labels_hbm, out_hbm, rows_v, lab_v, res_v, csum_v,
             all_v, out_v, shared):
  sid = lax.axis_index("s")
  iota = _iota()
  pltpu.sync_copy(labels_hbm, lab_v)
  pltpu.sync_copy(pred_hbm.at[pl.ds(sid * _RPS, _RPS)], rows_v)

  for j in range(_RPS):
    jv = _splat_i(j)
    r = sid * _RPS + j
    lab = jnp.max(plsc.load_gather(lab_v, [_splat_i(r)]))

    def _max_body(i, vm):
      for u in range(_UNROLL):
        x = plsc.load_gather(rows_v, [jv, i * (16 * _UNROLL) + u * 16 + iota])
        vm = jnp.maximum(vm, x)
      return vm

    vm = lax.fori_loop(0, _CHUNKS // _UNROLL, _max_body,
                       _splat_f(jnp.float32(-3.4e38)))
    m = jnp.max(vm)

    def _sum_body(i, vs):
      for u in range(_UNROLL):
        x = plsc.load_gather(rows_v, [jv, i * (16 * _UNROLL) + u * 16 + iota])
        vs = vs + jnp.exp(x - m)
      return vs

    vs = lax.fori_loop(0, _CHUNKS // _UNROLL, _sum_body,
                       jnp.zeros((16,), jnp.float32))
    s_full = jnp.sum(vs)
    xlv = plsc.load_gather(rows_v, [jv, _splat_i(lab)])
    x_l = jnp.max(xlv)
    e_l = jnp.max(jnp.exp(xlv - m))
    thresh = jnp.float32(0.001) * (s_full - e_l)

    def _sel_cond(c):
      chunk, cnt = c
      return jnp.logical_and(cnt < _TOPN, chunk < _CHUNKS)

    def _sel_body(c):
      chunk, cnt = c
      colv = chunk * 16 + iota
      x = plsc.load_gather(rows_v, [jv, colv])
      ev = jnp.exp(x - m)
      cond = jnp.logical_and(ev < thresh, colv != lab)
      ci = cond.astype(jnp.int32)
      posn = cnt + plsc.cumsum(ci) - 1
      plsc.store_scatter(res_v, [jv, posn], x,
                         mask=jnp.logical_and(cond, posn < _TOPN))
      return chunk + 1, cnt + jnp.sum(ci)

    _, cnt_f = lax.while_loop(_sel_cond, _sel_body,
                              (jnp.int32(0), jnp.int32(0)))

    padidx = jnp.where(lab == _C - 1, _C - 2, _C - 1)
    padv = plsc.load_gather(rows_v, [jv, _splat_i(padidx)])
    for k in range(_TOPN // 16):
      slot = k * 16 + iota
      old = res_v[j, pl.ds(k * 16, 16)]
      res_v[j, pl.ds(k * 16, 16)] = jnp.where(slot >= cnt_f, padv, old)
    meta = jnp.where(iota == 0, m,
                     jnp.where(iota == 1, s_full,
                               jnp.where(iota == 2, x_l, jnp.float32(0.0))))
    res_v[j, pl.ds(_TOPN, 16)] = meta

  pltpu.sync_copy(res_v, shared.at[pl.ds(sid * _RPS, _RPS)])
  plsc.subcore_barrier()

  @pl.when(sid == 0)
  def _final():
    pltpu.sync_copy(shared, all_v)
    carry = jnp.float32(0.0)
    for t in range(_B * _TOPN // 16):
      b, q = divmod(t, _TOPN // 16)
      x = all_v[b, pl.ds(q * 16, 16)]
      csum_v[pl.ds(t * 16, 16)] = plsc.cumsum(x) + carry
      carry = carry + jnp.sum(x)

    xm = []
    sum_xm = jnp.float32(0.0)
    for k in range(_B // 16):
      ivec = k * 16 + iota
      a = plsc.load_gather(csum_v, [65 * ivec])
      blo = plsc.load_gather(csum_v, [jnp.maximum(65 * ivec - 65, 0)])
      b_ = jnp.where(ivec == 0, jnp.float32(0.0), blo)
      xm.append((a - b_) * jnp.float32(1.0 / 65.0))
      sum_xm = sum_xm + jnp.sum(xm[k])

    cev = jnp.zeros((16,), jnp.float32)
    for k in range(_B // 16):
      rvec = k * 16 + iota
      mv = plsc.load_gather(all_v, [rvec, _splat_i(_TOPN + 0)])
      sv = plsc.load_gather(all_v, [rvec, _splat_i(_TOPN + 1)])
      xlv2 = plsc.load_gather(all_v, [rvec, _splat_i(_TOPN + 2)])
      cev = cev + (mv + _vlog(sv) - xlv2)
    ce = jnp.sum(cev) * jnp.float32(1.0 / _B)

    vm2 = jnp.maximum(jnp.maximum(xm[0], xm[1]), jnp.maximum(xm[2], xm[3]))
    m2 = jnp.max(vm2)
    s2 = jnp.float32(0.0)
    for k in range(_B // 16):
      s2 = s2 + jnp.sum(jnp.exp(xm[k] - m2))
    lse2 = m2 + jnp.max(_vlog(_splat_f(s2)))
    xloss = jnp.float32(_B) * lse2 - sum_xm
    loss = ce + jnp.float32(0.001) * xloss
    out_v[...] = _splat_f(loss)
    pltpu.sync_copy(out_v, out_hbm)


@jax.jit
def _run(pred, labels):
  mesh = plsc.VectorSubcoreMesh(
      core_axis_name="c", subcore_axis_name="s", num_cores=1,
      num_subcores=_NSUB)
  f = pl.kernel(
      _sc_body,
      out_type=jax.ShapeDtypeStruct((16,), jnp.float32),
      mesh=mesh,
      compiler_params=pltpu.CompilerParams(needs_layout_passes=False),
      scratch_types=[
          pltpu.VMEM((_RPS, _C), jnp.float32),
          pltpu.VMEM((_B,), jnp.int32),
          pltpu.VMEM((_RPS, _ROWW), jnp.float32),
          pltpu.VMEM((_B * _TOPN,), jnp.float32),
          pltpu.VMEM((_B, _ROWW), jnp.float32),
          pltpu.VMEM((16,), jnp.float32),
          pltpu.VMEM_SHARED((_B, _ROWW), jnp.float32),
      ],
  )
  return f(pred, labels)


def kernel(pred, labels):
  out = _run(pred, labels.astype(jnp.int32))
  return out[0]
